# Initial kernel scaffold; baseline (speedup 1.0000x reference)
import jax
import jax.numpy as jnp
from jax.experimental import pallas as pl


def kernel(feature_1, feature_2, table_1, table_2):
    raise NotImplementedError("write your pallas kernel here")


# speedup vs baseline: 143.9999x; 143.9999x over previous
import functools

import jax
import jax.numpy as jnp
from jax import lax
from jax.experimental import pallas as pl
from jax.experimental.pallas import tpu as pltpu
from jax.experimental.pallas import tpu_sc as plsc

B = 16384
L = 200
VOCAB = 10
DIM = 3

NUM_CORES = 2
NUM_SUBCORES = 16
NUM_WORKERS = NUM_CORES * NUM_SUBCORES

ROWS_PER_WORKER = B // NUM_WORKERS
SUBCHUNK = 128
NUM_SUBCHUNKS = ROWS_PER_WORKER // SUBCHUNK
GROUPS_PER_SUBCHUNK = SUBCHUNK // 16
L_UNROLL = 8

_mesh = plsc.VectorSubcoreMesh(core_axis_name="c", subcore_axis_name="s")


@functools.partial(
    pl.kernel,
    out_type=jax.ShapeDtypeStruct((B, DIM), jnp.float32),
    mesh=_mesh,
    scratch_types=[
        pltpu.VMEM((SUBCHUNK, L), jnp.int32),
        pltpu.VMEM((SUBCHUNK, L), jnp.int32),
        pltpu.VMEM((32,), jnp.float32),
        pltpu.VMEM((32,), jnp.float32),
        pltpu.VMEM((16, 16), jnp.float32),
        pltpu.VMEM((16, 16), jnp.float32),
        pltpu.VMEM((ROWS_PER_WORKER, DIM), jnp.float32),
    ],
    compiler_params=pltpu.CompilerParams(
        use_tc_tiling_on_sc=False, needs_layout_passes=False),
)
def _merger_kernel(f1_hbm, f2_hbm, t1_hbm, t2_hbm, out_hbm,
                   f1_v, f2_v, t1_v, t2_v, cnt1, cnt2, out_v):
    wid = lax.axis_index("s") * NUM_CORES + lax.axis_index("c")
    row_base = wid * ROWS_PER_WORKER

    pltpu.sync_copy(t1_hbm, t1_v)
    pltpu.sync_copy(t2_hbm, t2_v)

    t1_lo, t1_hi = t1_v[pl.ds(0, 16)], t1_v[pl.ds(16, 16)]
    t2_lo, t2_hi = t2_v[pl.ds(0, 16)], t2_v[pl.ds(16, 16)]

    def _entry(lo, hi, v, d):
        k = v * DIM + d
        return lo[k] if k < 16 else hi[k - 16]

    t1_s = [[_entry(t1_lo, t1_hi, v, d) for d in range(DIM)] for v in range(VOCAB)]
    t2_s = [[_entry(t2_lo, t2_hi, v, d) for d in range(DIM)] for v in range(VOCAB)]

    iota16 = lax.iota(jnp.int32, 16)
    ones16 = jnp.ones((16,), jnp.float32)
    zeros16 = jnp.zeros((16,), jnp.float32)
    inv_l = jnp.float32(1.0 / L)

    for c in range(NUM_SUBCHUNKS):
        row0 = row_base + c * SUBCHUNK
        pltpu.sync_copy(f1_hbm.at[pl.ds(row0, SUBCHUNK)], f1_v)
        pltpu.sync_copy(f2_hbm.at[pl.ds(row0, SUBCHUNK)], f2_v)

        def group_body(g, _, c=c):
            rows_g = g * 16 + iota16

            for r in range(16):
                cnt1[r, :] = zeros16
                cnt2[r, :] = zeros16

            def l_body(lb, _):
                for j in range(L_UNROLL):
                    lvec = jnp.full((16,), lb * L_UNROLL + j, jnp.int32)
                    i1 = plsc.load_gather(f1_v, [rows_g, lvec])
                    i2 = plsc.load_gather(f2_v, [rows_g, lvec])
                    plsc.addupdate_scatter(cnt1, [iota16, i1], ones16)
                    plsc.addupdate_scatter(cnt2, [iota16, i2], ones16)
                return 0

            lax.fori_loop(0, L // L_UNROLL, l_body, 0)

            acc = [zeros16, zeros16, zeros16]
            for v in range(VOCAB):
                vvec = jnp.full((16,), v, jnp.int32)
                c1 = plsc.load_gather(cnt1, [iota16, vvec])
                c2 = plsc.load_gather(cnt2, [iota16, vvec])
                for d in range(DIM):
                    acc[d] = acc[d] + c1 * t1_s[v][d] + c2 * t2_s[v][d]

            out_rows = c * SUBCHUNK + g * 16 + iota16
            for d in range(DIM):
                plsc.store_scatter(
                    out_v, [out_rows, jnp.full((16,), d, jnp.int32)],
                    acc[d] * inv_l)
            return 0

        lax.fori_loop(0, GROUPS_PER_SUBCHUNK, group_body, 0)

    pltpu.sync_copy(out_v, out_hbm.at[pl.ds(row_base, ROWS_PER_WORKER)])


def kernel(feature_1, feature_2, table_1, table_2):
    t1_flat = jnp.pad(table_1.reshape(-1), (0, 32 - VOCAB * DIM))
    t2_flat = jnp.pad(table_2.reshape(-1), (0, 32 - VOCAB * DIM))
    return _merger_kernel(feature_1, feature_2, t1_flat, t2_flat)
